# baseline (speedup 1.0000x reference)
import functools

import jax
import jax.numpy as jnp
from jax import lax
from jax.experimental import pallas as pl
from jax.experimental.pallas import tpu as pltpu
from jax.experimental.pallas import tpu_sc as plsc

B = 128
P = 2000
U = 1000
D = 256
PP = P + 1
SQRT_D = 16.0
CLIP = 10.0

NC = 2
NS = 16
NW = NC * NS
GTOT = 131072
GROWS = GTOT // 128
GPW = GROWS // NW
UPAD = 1008
WPAD = 2016
RW = 8

@functools.cache
def _sc_kernels():
    mesh = plsc.VectorSubcoreMesh(
        core_axis_name="c", subcore_axis_name="s",
        num_cores=NC, num_subcores=NS)

    params = pltpu.CompilerParams(use_tc_tiling_on_sc=False)

    @functools.partial(
        pl.kernel,
        out_type=jax.ShapeDtypeStruct((GROWS, 128, RW), jnp.float32),
        mesh=mesh,
        compiler_params=params,
        scratch_types=[
            pltpu.VMEM((GPW, 128), jnp.int32),
            pltpu.VMEM((128, RW), jnp.float32),
            pltpu.SemaphoreType.DMA,
        ],
    )
    def sc_gather(idx_hbm, table_hbm, out_hbm, idx_v, rows_v, sem):
        wid = lax.axis_index("s") * NC + lax.axis_index("c")
        base = wid * GPW
        pltpu.sync_copy(idx_hbm.at[pl.ds(base, GPW)], idx_v)

        def chunk(i, carry):
            pltpu.async_copy(table_hbm.at[idx_v.at[i]], rows_v, sem).wait()
            pltpu.sync_copy(rows_v, out_hbm.at[base + i])
            return carry

        lax.fori_loop(0, GPW, chunk, 0)

    @functools.partial(
        pl.kernel,
        out_type=jax.ShapeDtypeStruct((B, WPAD), jnp.float32),
        mesh=mesh,
        compiler_params=pltpu.CompilerParams(
            use_tc_tiling_on_sc=False, needs_layout_passes=False),
        scratch_types=[
            pltpu.VMEM((WPAD,), jnp.float32),
            pltpu.VMEM((UPAD,), jnp.int32),
            pltpu.VMEM((UPAD,), jnp.float32),
        ],
    )
    def sc_scatter(idx_hbm, p_hbm, out_hbm, buf, idx_v, p_v):
        wid = lax.axis_index("s") * NC + lax.axis_index("c")

        def row(r, carry):
            b = wid * (B // NW) + r

            def zero(i, c):
                buf[pl.ds(i * 16, 16)] = jnp.zeros((16,), jnp.float32)
                return c

            lax.fori_loop(0, WPAD // 16, zero, 0)
            pltpu.sync_copy(idx_hbm.at[b], idx_v)
            pltpu.sync_copy(p_hbm.at[b], p_v)

            def scat(i, c):
                iv = idx_v[pl.ds(i * 16, 16)]
                pv = p_v[pl.ds(i * 16, 16)]
                plsc.store_scatter(buf, [iv], pv)
                return c

            lax.fori_loop(0, UPAD // 16, scat, 0)
            pltpu.sync_copy(buf, out_hbm.at[b])
            return carry

        lax.fori_loop(0, B // NW, row, 0)

    return sc_gather, sc_scatter


RPB = 4


def _tc_body(dnd_ref, cur_ref, dist_ref, mask_ref, idx_ref, scal_ref,
             wemb_ref, bemb_ref, wqm_ref, wql_ref, wkv_ref,
             probs_ref, sel_ref, val_ref):
    wemb = wemb_ref[...]
    bemb = bemb_ref[...]
    scal = scal_ref[0]
    s_attn = scal[:, 0:1]
    s_com = scal[:, 1:2]

    for r in range(RPB):
        dnd = dnd_ref[r]
        enc_u = jnp.dot(dnd, wemb) + bemb
        kv = lax.dot_general(enc_u.astype(jnp.bfloat16), wkv_ref[...],
                             (((1,), (0,)), ((), ())),
                             preferred_element_type=jnp.float32)
        k = kv[:, :D]
        v = kv[:, D:]

        cur = cur_ref[r]
        enc_last = jnp.dot(cur, wemb) + bemb
        ld = cur[:, 3:4]
        q = jnp.dot(enc_last, wqm_ref[...]) + ld * wql_ref[...]
        sq = jax.nn.sigmoid(q)

        dist = dist_ref[r]
        mask = mask_ref[r]

        bias = -s_attn * dist + mask
        w = jnp.exp(bias - jnp.max(bias, axis=1, keepdims=True))
        k_exp = jnp.exp(k - jnp.max(k, axis=0, keepdims=True))
        num = lax.dot_general(w, k_exp * v, (((1,), (0,)), ((), ())))
        den = lax.dot_general(w, k_exp, (((1,), (0,)), ((), ())))
        aft = sq * (num / den)

        score = lax.dot_general(aft, enc_u, (((1,), (1,)), ((), ()))) / SQRT_D
        score = CLIP * jnp.tanh(score) - s_com * dist + mask
        m = jnp.max(score, axis=1, keepdims=True)
        e = jnp.exp(score - m)
        p = e / jnp.sum(e, axis=1, keepdims=True)
        probs_ref[r] = p

        pm = jnp.max(p, axis=1, keepdims=True)
        iota = lax.broadcasted_iota(jnp.int32, (1, U), 1)
        sel = jnp.min(jnp.where(p == pm, iota, U), axis=1, keepdims=True)
        tsel = jnp.sum(jnp.where(iota == sel, idx_ref[r], 0), axis=1,
                       keepdims=True)
        sel_ref[r] = jnp.broadcast_to(tsel, (1, 128))
        val_ref[r] = jnp.broadcast_to(pm, (1, 128))


def _tc_call(dnd_u, cur4, dist3, mask3, idx3, scal, wemb, bemb, wqm, wql,
             wkv):
    full = lambda shape: pl.BlockSpec(shape, lambda b: (0,) * len(shape))
    perb3 = lambda shape: pl.BlockSpec(shape, lambda b: (b, 0, 0))
    return pl.pallas_call(
        _tc_body,
        grid=(B // RPB,),
        in_specs=[
            perb3((RPB, U, RW)),
            perb3((RPB, 1, RW)),
            perb3((RPB, 1, U)),
            perb3((RPB, 1, U)),
            perb3((RPB, 1, U)),
            pl.BlockSpec((1, 1, 4), lambda b: (0, 0, 0)),
            full((RW, D)),
            full((1, D)),
            full((D, D)),
            full((1, D)),
            full((D, 2 * D)),
        ],
        out_specs=[
            perb3((RPB, 1, U)),
            perb3((RPB, 1, 128)),
            perb3((RPB, 1, 128)),
        ],
        out_shape=[
            jax.ShapeDtypeStruct((B, 1, U), jnp.float32),
            jax.ShapeDtypeStruct((B, 1, 128), jnp.int32),
            jax.ShapeDtypeStruct((B, 1, 128), jnp.float32),
        ],
    )(dnd_u, cur4, dist3, mask3, idx3, scal, wemb, bemb, wqm, wql, wkv)


@jax.jit
def kernel(depot_xy, node_xy, node_demand, load, cur_dist, ninf_mask,
           log_scale, W_emb, b_emb, Wq_last, Wk, Wv, alpha_attn, alpha_com,
           current_node, unvisited_index):
    depotr = jnp.concatenate(
        [depot_xy, jnp.zeros((B, 1, RW - 2), jnp.float32)], axis=2)
    noder = jnp.concatenate(
        [node_xy, node_demand[:, :, None],
         jnp.zeros((B, P, RW - 3), jnp.float32)], axis=2)
    table = jnp.concatenate([depotr, noder], axis=1).reshape(B * PP, RW)

    offs = (jnp.arange(B, dtype=jnp.int32) * PP)[:, None]
    uidx = unvisited_index.astype(jnp.int32)
    gidx = (uidx + offs).reshape(-1)
    cidx = current_node.astype(jnp.int32) + offs[:, 0]
    allidx = jnp.concatenate(
        [gidx, cidx, jnp.zeros((GTOT - B * U - B,), jnp.int32)])
    idx2d = allidx.reshape(GROWS, 128)

    sc_gather, sc_scatter = _sc_kernels()
    g = sc_gather(idx2d, table).reshape(GTOT, RW)
    dnd_u = g[: B * U].reshape(B, U, RW)
    currow = g[B * U: B * U + B]
    cur4 = jnp.concatenate(
        [currow[:, :3], load[:, None],
         jnp.zeros((B, RW - 4), jnp.float32)], axis=1).reshape(B, 1, RW)

    scal = jnp.stack([log_scale[0] * alpha_attn[0],
                      log_scale[0] * alpha_com[0],
                      jnp.float32(0.0), jnp.float32(0.0)]).reshape(1, 1, 4)
    wemb = jnp.concatenate([W_emb, jnp.zeros((RW - 3, D), jnp.float32)],
                           axis=0)
    wkv = jnp.concatenate([Wk, Wv], axis=1).astype(jnp.bfloat16)
    probs3, sel3, val3 = _tc_call(
        dnd_u, cur4, cur_dist.reshape(B, 1, U), ninf_mask.reshape(B, 1, U),
        uidx.reshape(B, 1, U), scal, wemb, b_emb.reshape(1, D),
        Wq_last[:D], Wq_last[D:D + 1], wkv)

    probs = probs3.reshape(B, U)
    idx_pad = jnp.concatenate(
        [uidx, jnp.full((B, UPAD - U), PP + 1, jnp.int32)], axis=1)
    p_pad = jnp.concatenate(
        [probs, jnp.zeros((B, UPAD - U), jnp.float32)], axis=1)
    upper = sc_scatter(idx_pad, p_pad)[:, :PP]

    return (upper, sel3[:, 0, 0], val3[:, 0, 0])


# speedup vs baseline: 1.2055x; 1.2055x over previous
import functools

import jax
import jax.numpy as jnp
from jax import lax
from jax.experimental import pallas as pl
from jax.experimental.pallas import tpu as pltpu
from jax.experimental.pallas import tpu_sc as plsc

B = 128
P = 2000
U = 1000
D = 256
PP = P + 1
SQRT_D = 16.0
CLIP = 10.0

NC = 2
NS = 16
NW = NC * NS
GTOT = 131072
GROWS = GTOT // 128
GPW = GROWS // NW
UPAD = 1008
WPAD = 2016
RW = 8
UW = 1024

@functools.cache
def _sc_kernels():
    mesh = plsc.VectorSubcoreMesh(
        core_axis_name="c", subcore_axis_name="s",
        num_cores=NC, num_subcores=NS)

    params = pltpu.CompilerParams(use_tc_tiling_on_sc=False)

    @functools.partial(
        pl.kernel,
        out_type=jax.ShapeDtypeStruct((GROWS, 128, RW), jnp.float32),
        mesh=mesh,
        compiler_params=params,
        scratch_types=[
            pltpu.VMEM((GPW, 128), jnp.int32),
            pltpu.VMEM((128, RW), jnp.float32),
            pltpu.SemaphoreType.DMA,
        ],
    )
    def sc_gather(idx_hbm, table_hbm, out_hbm, idx_v, rows_v, sem):
        wid = lax.axis_index("s") * NC + lax.axis_index("c")
        base = wid * GPW
        pltpu.sync_copy(idx_hbm.at[pl.ds(base, GPW)], idx_v)

        def chunk(i, carry):
            pltpu.async_copy(table_hbm.at[idx_v.at[i]], rows_v, sem).wait()
            pltpu.sync_copy(rows_v, out_hbm.at[base + i])
            return carry

        lax.fori_loop(0, GPW, chunk, 0)

    @functools.partial(
        pl.kernel,
        out_type=jax.ShapeDtypeStruct((B, WPAD), jnp.float32),
        mesh=mesh,
        compiler_params=pltpu.CompilerParams(
            use_tc_tiling_on_sc=False, needs_layout_passes=False),
        scratch_types=[
            pltpu.VMEM((WPAD,), jnp.float32),
            pltpu.VMEM((UPAD,), jnp.int32),
            pltpu.VMEM((UPAD,), jnp.float32),
        ],
    )
    def sc_scatter(idx_hbm, p_hbm, out_hbm, buf, idx_v, p_v):
        wid = lax.axis_index("s") * NC + lax.axis_index("c")
        tail_mask = lax.broadcasted_iota(jnp.int32, (16,), 0) < (U % 16)

        def row(r, carry):
            b = wid * (B // NW) + r

            def zero(i, c):
                buf[pl.ds(i * 16, 16)] = jnp.zeros((16,), jnp.float32)
                return c

            lax.fori_loop(0, WPAD // 16, zero, 0)
            pltpu.sync_copy(idx_hbm.at[b], idx_v.at[pl.ds(0, U)])
            pltpu.sync_copy(p_hbm.at[b], p_v.at[pl.ds(0, U)])

            def scat(i, c):
                iv = idx_v[pl.ds(i * 16, 16)]
                pv = p_v[pl.ds(i * 16, 16)]
                plsc.store_scatter(buf, [iv], pv)
                return c

            lax.fori_loop(0, U // 16, scat, 0)
            iv = idx_v[pl.ds((U // 16) * 16, 16)]
            pv = p_v[pl.ds((U // 16) * 16, 16)]
            plsc.store_scatter(buf, [iv], pv, mask=tail_mask)
            pltpu.sync_copy(buf, out_hbm.at[b])
            return carry

        lax.fori_loop(0, B // NW, row, 0)

    return sc_gather, sc_scatter


RPB = 4


def _tc_body(dnd_ref, dist_ref, mask_ref, idx_ref, scal_ref,
             wemb_ref, bemb_ref, wqm_ref, wql_ref, wkv_ref,
             probs_ref, sel_ref, val_ref):
    wemb = wemb_ref[...]
    bemb = bemb_ref[...]
    scal = scal_ref[0]
    s_attn = scal[:, 0:1]
    s_com = scal[:, 1:2]

    for r in range(RPB):
        dnd_all = dnd_ref[r]
        enc_all = jnp.dot(dnd_all, wemb) + bemb
        enc_u = enc_all[:U]
        kv = lax.dot_general(enc_u.astype(jnp.bfloat16), wkv_ref[...],
                             (((1,), (0,)), ((), ())),
                             preferred_element_type=jnp.float32)
        k = kv[:, :D]
        v = kv[:, D:]

        enc_last = enc_all[U:U + 1]
        ld = dnd_all[U:U + 1, 3:4]
        q = jnp.dot(enc_last, wqm_ref[...]) + ld * wql_ref[...]
        sq = jax.nn.sigmoid(q)

        dist = dist_ref[r]
        mask = mask_ref[r]

        bias = -s_attn * dist + mask
        w = jnp.exp(bias - jnp.max(bias, axis=1, keepdims=True))
        k_exp = jnp.exp(k - jnp.max(k, axis=0, keepdims=True))
        num = lax.dot_general(w, k_exp * v, (((1,), (0,)), ((), ())))
        den = lax.dot_general(w, k_exp, (((1,), (0,)), ((), ())))
        aft = sq * (num / den)

        score = lax.dot_general(aft, enc_u, (((1,), (1,)), ((), ()))) / SQRT_D
        score = CLIP * jnp.tanh(score) - s_com * dist + mask
        m = jnp.max(score, axis=1, keepdims=True)
        e = jnp.exp(score - m)
        p = e / jnp.sum(e, axis=1, keepdims=True)
        probs_ref[r] = p

        pm = jnp.max(p, axis=1, keepdims=True)
        iota = lax.broadcasted_iota(jnp.int32, (1, U), 1)
        sel = jnp.min(jnp.where(p == pm, iota, U), axis=1, keepdims=True)
        tsel = jnp.sum(jnp.where(iota == sel, idx_ref[r], 0), axis=1,
                       keepdims=True)
        sel_ref[r] = jnp.broadcast_to(tsel, (1, 128))
        val_ref[r] = jnp.broadcast_to(pm, (1, 128))


def _tc_call(dnd_u, dist3, mask3, idx3, scal, wemb, bemb, wqm, wql, wkv):
    full = lambda shape: pl.BlockSpec(shape, lambda b: (0,) * len(shape))
    perb3 = lambda shape: pl.BlockSpec(shape, lambda b: (b, 0, 0))
    return pl.pallas_call(
        _tc_body,
        grid=(B // RPB,),
        in_specs=[
            perb3((RPB, UW, RW)),
            perb3((RPB, 1, U)),
            perb3((RPB, 1, U)),
            perb3((RPB, 1, U)),
            pl.BlockSpec((1, 1, 4), lambda b: (0, 0, 0)),
            full((RW, D)),
            full((1, D)),
            full((D, D)),
            full((1, D)),
            full((D, 2 * D)),
        ],
        out_specs=[
            perb3((RPB, 1, U)),
            perb3((RPB, 1, 128)),
            perb3((RPB, 1, 128)),
        ],
        out_shape=[
            jax.ShapeDtypeStruct((B, 1, U), jnp.float32),
            jax.ShapeDtypeStruct((B, 1, 128), jnp.int32),
            jax.ShapeDtypeStruct((B, 1, 128), jnp.float32),
        ],
    )(dnd_u, dist3, mask3, idx3, scal, wemb, bemb, wqm, wql, wkv)


@jax.jit
def kernel(depot_xy, node_xy, node_demand, load, cur_dist, ninf_mask,
           log_scale, W_emb, b_emb, Wq_last, Wk, Wv, alpha_attn, alpha_com,
           current_node, unvisited_index):
    loadc = jnp.broadcast_to(load[:, None, None], (B, PP, 1))
    depotr = jnp.concatenate(
        [depot_xy, jnp.zeros((B, 1, 1), jnp.float32), loadc[:, :1],
         jnp.zeros((B, 1, RW - 4), jnp.float32)], axis=2)
    noder = jnp.concatenate(
        [node_xy, node_demand[:, :, None], loadc[:, 1:],
         jnp.zeros((B, P, RW - 4), jnp.float32)], axis=2)
    table = jnp.concatenate([depotr, noder], axis=1).reshape(B * PP, RW)

    offs = (jnp.arange(B, dtype=jnp.int32) * PP)[:, None]
    uidx = unvisited_index.astype(jnp.int32)
    cidx = current_node.astype(jnp.int32)[:, None] + offs
    rows = jnp.concatenate(
        [uidx + offs, cidx, jnp.zeros((B, UW - U - 1), jnp.int32)], axis=1)
    idx2d = rows.reshape(GROWS, 128)

    sc_gather, sc_scatter = _sc_kernels()
    dnd_u = sc_gather(idx2d, table).reshape(B, UW, RW)

    scal = jnp.stack([log_scale[0] * alpha_attn[0],
                      log_scale[0] * alpha_com[0],
                      jnp.float32(0.0), jnp.float32(0.0)]).reshape(1, 1, 4)
    wemb = jnp.concatenate([W_emb, jnp.zeros((RW - 3, D), jnp.float32)],
                           axis=0)
    wkv = jnp.concatenate([Wk, Wv], axis=1).astype(jnp.bfloat16)
    probs3, sel3, val3 = _tc_call(
        dnd_u, cur_dist.reshape(B, 1, U), ninf_mask.reshape(B, 1, U),
        uidx.reshape(B, 1, U), scal, wemb, b_emb.reshape(1, D),
        Wq_last[:D], Wq_last[D:D + 1], wkv)

    upper = sc_scatter(uidx, probs3.reshape(B, U))[:, :PP]

    return (upper, sel3[:, 0, 0], val3[:, 0, 0])
